# baseline (speedup 1.0000x reference)
import functools

import jax
import jax.numpy as jnp
from jax import lax
from jax.experimental import pallas as pl
from jax.experimental.pallas import tpu as pltpu
from jax.experimental.pallas import tpu_sc as plsc

_N = 10000
_NPAD = 10240
_E = 160000
_CH = _E // 128
_NS = 16
_NT = (_CH + _NS - 1) // _NS
_W = _NPAD // _NS
_NB = _NPAD // 512


def _sc_mesh():
    return plsc.VectorSubcoreMesh(core_axis_name="c", subcore_axis_name="s")


def _sc_degrees(s1, d1, s2, d2):
    out = jax.ShapeDtypeStruct((_NPAD,), jnp.float32)

    @functools.partial(
        pl.kernel,
        out_type=(out, out, out, out),
        mesh=_sc_mesh(),
        scratch_types=[
            pltpu.VMEM((1, 128), jnp.int32),
            pltpu.VMEM((1, 128), jnp.int32),
            pltpu.VMEM((128,), jnp.float32),
            pltpu.VMEM((_W,), jnp.float32),
            pltpu.VMEM_SHARED((_NPAD,), jnp.float32),
            pltpu.VMEM_SHARED((_NPAD,), jnp.float32),
        ],
    )
    def k(s1h, d1h, s2h, d2h, od1, id1, od2, id2, sv, dv, ones_v, zv, hs, hd):
        c = lax.axis_index("c")
        s = lax.axis_index("s")
        for u in range(8):
            ones_v[pl.ds(u * 16, 16)] = jnp.ones((16,), jnp.float32)

        def zf(i, _):
            zv[pl.ds(i * 16, 16)] = jnp.zeros((16,), jnp.float32)
            return 0

        lax.fori_loop(0, _W // 16, zf, 0)
        pltpu.sync_copy(zv, hs.at[pl.ds(s * _W, _W)])
        pltpu.sync_copy(zv, hd.at[pl.ds(s * _W, _W)])
        plsc.subcore_barrier()

        def run(src_hbm, dst_hbm):
            def body(t, _):
                cid = s + t * _NS

                @pl.when(cid < _CH)
                def _():
                    pltpu.sync_copy(src_hbm.at[pl.ds(cid, 1)], sv)
                    pltpu.sync_copy(dst_hbm.at[pl.ds(cid, 1)], dv)
                    pltpu.sync_copy(ones_v, hs.at[sv.at[0]], add=True)
                    pltpu.sync_copy(ones_v, hd.at[dv.at[0]], add=True)

                return 0

            lax.fori_loop(0, _NT, body, 0)

        @pl.when(c == 0)
        def _():
            run(s1h, d1h)

        @pl.when(c == 1)
        def _():
            run(s2h, d2h)

        plsc.subcore_barrier()
        sl = pl.ds(s * _W, _W)

        @pl.when(c == 0)
        def _():
            pltpu.sync_copy(hs.at[sl], od1.at[sl])
            pltpu.sync_copy(hd.at[sl], id1.at[sl])

        @pl.when(c == 1)
        def _():
            pltpu.sync_copy(hs.at[sl], od2.at[sl])
            pltpu.sync_copy(hd.at[sl], id2.at[sl])

    return k(s1, d1, s2, d2)


def _sc_segsum(x1r, x2r, s1, d1, s2, d2):
    out = jax.ShapeDtypeStruct((_NPAD, 128), jnp.float32)

    nbuf = 2

    @functools.partial(
        pl.kernel,
        out_type=(out, out, out, out),
        mesh=_sc_mesh(),
        scratch_types=[
            pltpu.VMEM((8, 128), jnp.int32),
            pltpu.VMEM((8, 128), jnp.int32),
            [pltpu.VMEM((128,), jnp.int32) for _ in range(nbuf)],
            [pltpu.VMEM((128, 128), jnp.float32) for _ in range(nbuf)],
            pltpu.VMEM((64, 128), jnp.float32),
            pltpu.VMEM_SHARED((_NPAD, 128), jnp.float32),
            pltpu.SemaphoreType.DMA,
        ],
    )
    def k(x1h, x2h, s1h, d1h, s2h, d2h, o1lo, o1hi, o2lo, o2hi,
          sv8, dv8, gvs, rbs, zb, acc, gsem):
        c = lax.axis_index("c")
        s = lax.axis_index("s")
        base = s * 80
        nt = jnp.minimum(80, _CH - base)

        def zf(i, _):
            for u in range(8):
                zb[i, pl.ds(u * 16, 16)] = jnp.zeros((16,), jnp.float32)
            return 0

        lax.fori_loop(0, 64, zf, 0)

        def one_view(x_hbm, src_hbm, dst_hbm, out_lo, out_hi):
            for kk in range(_W // 64):
                pltpu.sync_copy(zb, acc.at[pl.ds(s * _W + kk * 64, 64)])
            plsc.subcore_barrier()

            def body(o, _):
                o8 = base + o * 8
                pltpu.sync_copy(src_hbm.at[pl.ds(o8, 8)], sv8)
                pltpu.sync_copy(dst_hbm.at[pl.ds(o8, 8)], dv8)
                for gp in range(4):
                    ks = [gp * nbuf + b for b in range(nbuf)]
                    for b in range(nbuf):
                        @pl.when(o * 8 + ks[b] < nt)
                        def _(b=b, kk=ks[b]):
                            for u in range(8):
                                gvs[b][pl.ds(u * 16, 16)] = (
                                    sv8[kk, pl.ds(u * 16, 16)] * 2 + c)
                            pltpu.async_copy(x_hbm.at[gvs[b]], rbs[b], gsem)
                    for b in range(nbuf):
                        @pl.when(o * 8 + ks[b] < nt)
                        def _(b=b, kk=ks[b]):
                            pltpu.make_async_copy(
                                x_hbm.at[gvs[b]], rbs[b], gsem).wait()
                            pltpu.sync_copy(
                                rbs[b], acc.at[dv8.at[kk]], add=True)
                return 0

            lax.fori_loop(0, 10, body, 0)
            plsc.subcore_barrier()
            sl = pl.ds(s * _W, _W)

            @pl.when(c == 0)
            def _():
                pltpu.sync_copy(acc.at[sl], out_lo.at[sl])

            @pl.when(c == 1)
            def _():
                pltpu.sync_copy(acc.at[sl], out_hi.at[sl])

            plsc.subcore_barrier()

        one_view(x1h, s1h, d1h, o1lo, o1hi)
        one_view(x2h, s2h, d2h, o2lo, o2hi)

    return k(x1r, x2r, s1, d1, s2, d2)


def _tc_scale(h, odeg):
    def body(h_ref, od_ref, o_ref):
        scale = lax.rsqrt(jnp.maximum(od_ref[...], 1.0))
        o_ref[...] = h_ref[...] * scale

    return pl.pallas_call(
        body,
        grid=(10,),
        in_specs=[
            pl.BlockSpec((1000, 256), lambda i: (i, 0)),
            pl.BlockSpec((1000, 1), lambda i: (i, 0)),
        ],
        out_specs=pl.BlockSpec((1000, 256), lambda i: (i, 0)),
        out_shape=jax.ShapeDtypeStruct((_N, 256), jnp.float32),
    )(h, odeg)


def _tc_conv01(plo, phi, ideg, odeg, W0, b0, W1):
    def body(plo_ref, phi_ref, id_ref, od_ref, w0_ref, b0_ref, w1_ref, o_ref):
        a = jnp.dot(plo_ref[...], w0_ref[0:128, :],
                    preferred_element_type=jnp.float32)
        a = a + jnp.dot(phi_ref[...], w0_ref[128:256, :],
                        preferred_element_type=jnp.float32)
        iscale = lax.rsqrt(jnp.maximum(id_ref[...], 1.0))
        h = jnp.maximum(a * iscale + b0_ref[...], 0.0)
        oscale = lax.rsqrt(jnp.maximum(od_ref[...], 1.0))
        o_ref[...] = jnp.dot(h * oscale, w1_ref[...],
                             preferred_element_type=jnp.float32)

    return pl.pallas_call(
        body,
        grid=(_NB,),
        in_specs=[
            pl.BlockSpec((512, 128), lambda i: (i, 0)),
            pl.BlockSpec((512, 128), lambda i: (i, 0)),
            pl.BlockSpec((512, 1), lambda i: (i, 0)),
            pl.BlockSpec((512, 1), lambda i: (i, 0)),
            pl.BlockSpec((256, 512), lambda i: (0, 0)),
            pl.BlockSpec((1, 512), lambda i: (0, 0)),
            pl.BlockSpec((512, 256), lambda i: (0, 0)),
        ],
        out_specs=pl.BlockSpec((512, 256), lambda i: (i, 0)),
        out_shape=jax.ShapeDtypeStruct((_NPAD, 256), jnp.float32),
    )(plo, phi, ideg, odeg, W0, b0, W1)


def _tc_proj(qlo, qhi, ideg, b1, fc1_w, fc1_b, fc2_w, fc2_b):
    def body(qlo_ref, qhi_ref, id_ref, b1_ref, w1_ref, c1_ref, w2_ref, c2_ref,
             ob_ref, o2b_ref):
        i = pl.program_id(0)
        p = jnp.concatenate([qlo_ref[...], qhi_ref[...]], axis=1)
        iscale = lax.rsqrt(jnp.maximum(id_ref[...], 1.0))
        e = jnp.maximum(p * iscale + b1_ref[...], 0.0)
        t = jnp.dot(e, w1_ref[...], preferred_element_type=jnp.float32)
        t = t + c1_ref[...]
        t = jnp.where(t > 0, t, jnp.exp(t) - 1.0)
        z = jnp.dot(t, w2_ref[...], preferred_element_type=jnp.float32)
        z = z + c2_ref[...]
        nrm = jnp.sqrt(jnp.sum(z * z, axis=1, keepdims=True))
        n = z / jnp.maximum(nrm, 1e-12)
        row = i * 512 + lax.broadcasted_iota(jnp.int32, (512, 1), 0)
        n = jnp.where(row < _N, n, 0.0)
        ob_ref[...] = n.astype(jnp.bfloat16)
        o2b_ref[...] = (n * 2.8853900817779268).astype(jnp.bfloat16)

    return pl.pallas_call(
        body,
        grid=(_NB,),
        in_specs=[
            pl.BlockSpec((512, 128), lambda i: (i, 0)),
            pl.BlockSpec((512, 128), lambda i: (i, 0)),
            pl.BlockSpec((512, 1), lambda i: (i, 0)),
            pl.BlockSpec((1, 256), lambda i: (0, 0)),
            pl.BlockSpec((256, 128), lambda i: (0, 0)),
            pl.BlockSpec((1, 128), lambda i: (0, 0)),
            pl.BlockSpec((128, 256), lambda i: (0, 0)),
            pl.BlockSpec((1, 256), lambda i: (0, 0)),
        ],
        out_specs=[
            pl.BlockSpec((512, 256), lambda i: (i, 0)),
            pl.BlockSpec((512, 256), lambda i: (i, 0)),
        ],
        out_shape=(
            jax.ShapeDtypeStruct((_NPAD, 256), jnp.bfloat16),
            jax.ShapeDtypeStruct((_NPAD, 256), jnp.bfloat16),
        ),
    )(qlo, qhi, ideg, b1, fc1_w, fc1_b, fc2_w, fc2_b)


def _dot_t(x, y):
    return lax.dot_general(
        x, y, (((1,), (1,)), ((), ())), preferred_element_type=jnp.float32)


def _dot_n(x, y):
    return lax.dot_general(
        x, y, (((1,), (0,)), ((), ())), preferred_element_type=jnp.float32)


def _tc_loss(n1b, n1s, n2b, n2s):
    padc = float(_NPAD - _N)

    def body(n1s_j, n1b_i, n2s_j, n2b_i,
             out_ref, r11, r12, r22, c12, d11, d22, d12):
        i = pl.program_id(0)
        j = pl.program_id(1)

        ri = pl.ds(i, 1)
        rj = pl.ds(j, 1)

        @pl.when((i == 0) & (j == 0))
        def _():
            out_ref[...] = jnp.zeros((1, 1), jnp.float32)
            z = jnp.zeros((_NB, 512), jnp.float32)
            r11[...] = z
            r12[...] = z
            r22[...] = z
            c12[...] = z

        ones_r = jnp.ones((8, 512), jnp.bfloat16)

        e12t = jnp.exp2(_dot_t(n2s_j[...], n1b_i[...])).astype(jnp.bfloat16)
        r12[ri, :] = r12[ri, :] + _dot_n(ones_r, e12t)[0:1, :]
        c12[rj, :] = c12[rj, :] + _dot_t(ones_r, e12t)[0:1, :]

        @pl.when(j >= i)
        def _():
            e11t = jnp.exp2(
                _dot_t(n1s_j[...], n1b_i[...])).astype(jnp.bfloat16)
            e22t = jnp.exp2(
                _dot_t(n2s_j[...], n2b_i[...])).astype(jnp.bfloat16)
            r11[ri, :] = r11[ri, :] + _dot_n(ones_r, e11t)[0:1, :]
            r22[ri, :] = r22[ri, :] + _dot_n(ones_r, e22t)[0:1, :]

            @pl.when(j > i)
            def _():
                r11[rj, :] = r11[rj, :] + _dot_t(ones_r, e11t)[0:1, :]
                r22[rj, :] = r22[rj, :] + _dot_t(ones_r, e22t)[0:1, :]

            @pl.when(i == j)
            def _():
                dm = (lax.broadcasted_iota(jnp.int32, (512, 512), 0) ==
                      lax.broadcasted_iota(jnp.int32, (512, 512), 1))
                zb16 = jnp.bfloat16(0.0)
                d11[ri, :] = _dot_n(
                    ones_r, jnp.where(dm, e11t, zb16))[0:1, :]
                d12[ri, :] = _dot_n(
                    ones_r, jnp.where(dm, e12t, zb16))[0:1, :]
                d22[ri, :] = _dot_n(
                    ones_r, jnp.where(dm, e22t, zb16))[0:1, :]

        @pl.when((i == _NB - 1) & (j == _NB - 1))
        def _():
            rowmask = (lax.broadcasted_iota(jnp.int32, (_NB, 512), 0) * 512 +
                       lax.broadcasted_iota(jnp.int32, (_NB, 512), 1)) < _N
            x1 = (r11[...] - padc) + (r12[...] - padc) - d11[...]
            x2 = (r22[...] - padc) + (c12[...] - padc) - d22[...]
            li = -jnp.log(d12[...]) + 0.5 * (jnp.log(x1) + jnp.log(x2))
            li = jnp.where(rowmask, li, 0.0)
            out_ref[...] = (jnp.sum(li) / _N).reshape(1, 1)

    return pl.pallas_call(
        body,
        grid=(_NB, _NB),
        in_specs=[
            pl.BlockSpec((512, 256), lambda i, j: (j, 0)),
            pl.BlockSpec((512, 256), lambda i, j: (i, 0)),
            pl.BlockSpec((512, 256), lambda i, j: (j, 0)),
            pl.BlockSpec((512, 256), lambda i, j: (i, 0)),
        ],
        out_specs=pl.BlockSpec((1, 1), lambda i, j: (0, 0)),
        out_shape=jax.ShapeDtypeStruct((1, 1), jnp.float32),
        scratch_shapes=[pltpu.VMEM((_NB, 512), jnp.float32)
                        for _ in range(7)],
    )(n1s, n1b, n2s, n2b)


def kernel(h1, h2, g1, g2, W0, b0, W1, b1, fc1_w, fc1_b, fc2_w, fc2_b):
    s1 = g1[0].reshape(_CH, 128)
    d1 = g1[1].reshape(_CH, 128)
    s2 = g2[0].reshape(_CH, 128)
    d2 = g2[1].reshape(_CH, 128)
    pad = ((0, 30), (0, 0))
    s1p, d1p = jnp.pad(s1, pad), jnp.pad(d1, pad)
    s2p, d2p = jnp.pad(s2, pad), jnp.pad(d2, pad)

    od1, id1, od2, id2 = _sc_degrees(s1, d1, s2, d2)
    od1n = od1[:_N].reshape(_N, 1)
    od2n = od2[:_N].reshape(_N, 1)
    od1p = od1.reshape(_NPAD, 1)
    od2p = od2.reshape(_NPAD, 1)
    id1p = id1.reshape(_NPAD, 1)
    id2p = id2.reshape(_NPAD, 1)

    xs1 = _tc_scale(h1, od1n)
    xs2 = _tc_scale(h2, od2n)

    p1lo, p1hi, p2lo, p2hi = _sc_segsum(
        xs1.reshape(2 * _N, 128), xs2.reshape(2 * _N, 128),
        s1p, d1p, s2p, d2p)

    b0r = b0.reshape(1, 2 * 256)
    m1 = _tc_conv01(p1lo, p1hi, id1p, od1p, W0, b0r, W1)
    m2 = _tc_conv01(p2lo, p2hi, id2p, od2p, W0, b0r, W1)

    q1lo, q1hi, q2lo, q2hi = _sc_segsum(
        m1.reshape(2 * _NPAD, 128), m2.reshape(2 * _NPAD, 128),
        s1p, d1p, s2p, d2p)

    b1r = b1.reshape(1, 256)
    c1r = fc1_b.reshape(1, 128)
    c2r = fc2_b.reshape(1, 256)
    n1b, n1s = _tc_proj(q1lo, q1hi, id1p, b1r, fc1_w, c1r, fc2_w, c2r)
    n2b, n2s = _tc_proj(q2lo, q2hi, id2p, b1r, fc1_w, c1r, fc2_w, c2r)

    res = _tc_loss(n1b, n1s, n2b, n2s)
    return res.reshape(())


# speedup vs baseline: 1.0328x; 1.0328x over previous
import functools

import jax
import jax.numpy as jnp
from jax import lax
from jax.experimental import pallas as pl
from jax.experimental.pallas import tpu as pltpu
from jax.experimental.pallas import tpu_sc as plsc

_N = 10000
_NPAD = 10240
_E = 160000
_CH = _E // 128
_NS = 16
_NT = (_CH + _NS - 1) // _NS
_W = _NPAD // _NS
_NB = _NPAD // 512


def _sc_mesh():
    return plsc.VectorSubcoreMesh(core_axis_name="c", subcore_axis_name="s")


def _sc_degrees(s1, d1, s2, d2):
    out = jax.ShapeDtypeStruct((_NPAD,), jnp.float32)

    @functools.partial(
        pl.kernel,
        out_type=(out, out, out, out),
        mesh=_sc_mesh(),
        scratch_types=[
            pltpu.VMEM((1, 128), jnp.int32),
            pltpu.VMEM((1, 128), jnp.int32),
            pltpu.VMEM((128,), jnp.float32),
            pltpu.VMEM((_W,), jnp.float32),
            pltpu.VMEM_SHARED((_NPAD,), jnp.float32),
            pltpu.VMEM_SHARED((_NPAD,), jnp.float32),
        ],
    )
    def k(s1h, d1h, s2h, d2h, od1, id1, od2, id2, sv, dv, ones_v, zv, hs, hd):
        c = lax.axis_index("c")
        s = lax.axis_index("s")
        for u in range(8):
            ones_v[pl.ds(u * 16, 16)] = jnp.ones((16,), jnp.float32)

        def zf(i, _):
            zv[pl.ds(i * 16, 16)] = jnp.zeros((16,), jnp.float32)
            return 0

        lax.fori_loop(0, _W // 16, zf, 0)
        pltpu.sync_copy(zv, hs.at[pl.ds(s * _W, _W)])
        pltpu.sync_copy(zv, hd.at[pl.ds(s * _W, _W)])
        plsc.subcore_barrier()

        def run(src_hbm, dst_hbm):
            def body(t, _):
                cid = s + t * _NS

                @pl.when(cid < _CH)
                def _():
                    pltpu.sync_copy(src_hbm.at[pl.ds(cid, 1)], sv)
                    pltpu.sync_copy(dst_hbm.at[pl.ds(cid, 1)], dv)
                    pltpu.sync_copy(ones_v, hs.at[sv.at[0]], add=True)
                    pltpu.sync_copy(ones_v, hd.at[dv.at[0]], add=True)

                return 0

            lax.fori_loop(0, _NT, body, 0)

        @pl.when(c == 0)
        def _():
            run(s1h, d1h)

        @pl.when(c == 1)
        def _():
            run(s2h, d2h)

        plsc.subcore_barrier()
        sl = pl.ds(s * _W, _W)

        @pl.when(c == 0)
        def _():
            pltpu.sync_copy(hs.at[sl], od1.at[sl])
            pltpu.sync_copy(hd.at[sl], id1.at[sl])

        @pl.when(c == 1)
        def _():
            pltpu.sync_copy(hs.at[sl], od2.at[sl])
            pltpu.sync_copy(hd.at[sl], id2.at[sl])

    return k(s1, d1, s2, d2)


def _sc_segsum(x1r, x2r, s1, d1, s2, d2):
    out = jax.ShapeDtypeStruct((_NPAD, 128), jnp.float32)

    nbuf = 2

    @functools.partial(
        pl.kernel,
        out_type=(out, out, out, out),
        mesh=_sc_mesh(),
        scratch_types=[
            pltpu.VMEM((8, 128), jnp.int32),
            pltpu.VMEM((8, 128), jnp.int32),
            [pltpu.VMEM((128,), jnp.int32) for _ in range(nbuf)],
            [pltpu.VMEM((128, 128), jnp.float32) for _ in range(nbuf)],
            pltpu.VMEM((64, 128), jnp.float32),
            pltpu.VMEM_SHARED((_NPAD, 128), jnp.float32),
            pltpu.SemaphoreType.DMA,
        ],
    )
    def k(x1h, x2h, s1h, d1h, s2h, d2h, o1lo, o1hi, o2lo, o2hi,
          sv8, dv8, gvs, rbs, zb, acc, gsem):
        c = lax.axis_index("c")
        s = lax.axis_index("s")
        base = s * 80
        nt = jnp.minimum(80, _CH - base)

        def zf(i, _):
            for u in range(8):
                zb[i, pl.ds(u * 16, 16)] = jnp.zeros((16,), jnp.float32)
            return 0

        lax.fori_loop(0, 64, zf, 0)

        def one_view(x_hbm, src_hbm, dst_hbm, out_lo, out_hi):
            for kk in range(_W // 64):
                pltpu.sync_copy(zb, acc.at[pl.ds(s * _W + kk * 64, 64)])
            plsc.subcore_barrier()

            def body(o, _):
                o8 = base + o * 8
                pltpu.sync_copy(src_hbm.at[pl.ds(o8, 8)], sv8)
                pltpu.sync_copy(dst_hbm.at[pl.ds(o8, 8)], dv8)
                for gp in range(4):
                    ks = [gp * nbuf + b for b in range(nbuf)]
                    for b in range(nbuf):
                        @pl.when(o * 8 + ks[b] < nt)
                        def _(b=b, kk=ks[b]):
                            for u in range(8):
                                gvs[b][pl.ds(u * 16, 16)] = (
                                    sv8[kk, pl.ds(u * 16, 16)] * 2 + c)
                            pltpu.async_copy(x_hbm.at[gvs[b]], rbs[b], gsem)
                    for b in range(nbuf):
                        @pl.when(o * 8 + ks[b] < nt)
                        def _(b=b, kk=ks[b]):
                            pltpu.make_async_copy(
                                x_hbm.at[gvs[b]], rbs[b], gsem).wait()
                            pltpu.sync_copy(
                                rbs[b], acc.at[dv8.at[kk]], add=True)
                return 0

            lax.fori_loop(0, 10, body, 0)
            plsc.subcore_barrier()
            sl = pl.ds(s * _W, _W)

            @pl.when(c == 0)
            def _():
                pltpu.sync_copy(acc.at[sl], out_lo.at[sl])

            @pl.when(c == 1)
            def _():
                pltpu.sync_copy(acc.at[sl], out_hi.at[sl])

            plsc.subcore_barrier()

        one_view(x1h, s1h, d1h, o1lo, o1hi)
        one_view(x2h, s2h, d2h, o2lo, o2hi)

    return k(x1r, x2r, s1, d1, s2, d2)


def _tc_scale(h, odeg):
    def body(h_ref, od_ref, o_ref):
        scale = lax.rsqrt(jnp.maximum(od_ref[...], 1.0))
        o_ref[...] = h_ref[...] * scale

    return pl.pallas_call(
        body,
        grid=(10,),
        in_specs=[
            pl.BlockSpec((1000, 256), lambda i: (i, 0)),
            pl.BlockSpec((1000, 1), lambda i: (i, 0)),
        ],
        out_specs=pl.BlockSpec((1000, 256), lambda i: (i, 0)),
        out_shape=jax.ShapeDtypeStruct((_N, 256), jnp.float32),
    )(h, odeg)


def _tc_conv01(plo, phi, ideg, odeg, W0, b0, W1):
    def body(plo_ref, phi_ref, id_ref, od_ref, w0_ref, b0_ref, w1_ref, o_ref):
        a = jnp.dot(plo_ref[...], w0_ref[0:128, :],
                    preferred_element_type=jnp.float32)
        a = a + jnp.dot(phi_ref[...], w0_ref[128:256, :],
                        preferred_element_type=jnp.float32)
        iscale = lax.rsqrt(jnp.maximum(id_ref[...], 1.0))
        h = jnp.maximum(a * iscale + b0_ref[...], 0.0)
        oscale = lax.rsqrt(jnp.maximum(od_ref[...], 1.0))
        o_ref[...] = jnp.dot(h * oscale, w1_ref[...],
                             preferred_element_type=jnp.float32)

    return pl.pallas_call(
        body,
        grid=(_NB,),
        in_specs=[
            pl.BlockSpec((512, 128), lambda i: (i, 0)),
            pl.BlockSpec((512, 128), lambda i: (i, 0)),
            pl.BlockSpec((512, 1), lambda i: (i, 0)),
            pl.BlockSpec((512, 1), lambda i: (i, 0)),
            pl.BlockSpec((256, 512), lambda i: (0, 0)),
            pl.BlockSpec((1, 512), lambda i: (0, 0)),
            pl.BlockSpec((512, 256), lambda i: (0, 0)),
        ],
        out_specs=pl.BlockSpec((512, 256), lambda i: (i, 0)),
        out_shape=jax.ShapeDtypeStruct((_NPAD, 256), jnp.float32),
    )(plo, phi, ideg, odeg, W0, b0, W1)


def _tc_proj(qlo, qhi, ideg, b1, fc1_w, fc1_b, fc2_w, fc2_b):
    def body(qlo_ref, qhi_ref, id_ref, b1_ref, w1_ref, c1_ref, w2_ref, c2_ref,
             ob_ref, o2b_ref):
        i = pl.program_id(0)
        p = jnp.concatenate([qlo_ref[...], qhi_ref[...]], axis=1)
        iscale = lax.rsqrt(jnp.maximum(id_ref[...], 1.0))
        e = jnp.maximum(p * iscale + b1_ref[...], 0.0)
        t = jnp.dot(e, w1_ref[...], preferred_element_type=jnp.float32)
        t = t + c1_ref[...]
        t = jnp.where(t > 0, t, jnp.exp(t) - 1.0)
        z = jnp.dot(t, w2_ref[...], preferred_element_type=jnp.float32)
        z = z + c2_ref[...]
        nrm = jnp.sqrt(jnp.sum(z * z, axis=1, keepdims=True))
        n = z / jnp.maximum(nrm, 1e-12)
        row = i * 512 + lax.broadcasted_iota(jnp.int32, (512, 1), 0)
        n = jnp.where(row < _N, n, 0.0)
        ob_ref[...] = n.astype(jnp.bfloat16)
        o2b_ref[...] = (n * 2.8853900817779268).astype(jnp.bfloat16)

    return pl.pallas_call(
        body,
        grid=(_NB,),
        in_specs=[
            pl.BlockSpec((512, 128), lambda i: (i, 0)),
            pl.BlockSpec((512, 128), lambda i: (i, 0)),
            pl.BlockSpec((512, 1), lambda i: (i, 0)),
            pl.BlockSpec((1, 256), lambda i: (0, 0)),
            pl.BlockSpec((256, 128), lambda i: (0, 0)),
            pl.BlockSpec((1, 128), lambda i: (0, 0)),
            pl.BlockSpec((128, 256), lambda i: (0, 0)),
            pl.BlockSpec((1, 256), lambda i: (0, 0)),
        ],
        out_specs=[
            pl.BlockSpec((512, 256), lambda i: (i, 0)),
            pl.BlockSpec((512, 256), lambda i: (i, 0)),
        ],
        out_shape=(
            jax.ShapeDtypeStruct((_NPAD, 256), jnp.bfloat16),
            jax.ShapeDtypeStruct((_NPAD, 256), jnp.bfloat16),
        ),
    )(qlo, qhi, ideg, b1, fc1_w, fc1_b, fc2_w, fc2_b)


def _dot_t(x, y):
    return lax.dot_general(
        x, y, (((1,), (1,)), ((), ())), preferred_element_type=jnp.float32)


def _dot_n(x, y):
    return lax.dot_general(
        x, y, (((1,), (0,)), ((), ())), preferred_element_type=jnp.float32)


def _tc_loss(n1b, n1s, n2b, n2s):
    padc = float(_NPAD - _N)

    def body(n1s_j, n1b_i, n2s_j, n2b_i,
             out_ref, r11, r12, r22, c12, d11, d22, d12):
        i = pl.program_id(0)
        j = pl.program_id(1)

        ri = pl.ds(i, 1)
        rj = pl.ds(j, 1)

        @pl.when((i == 0) & (j == 0))
        def _():
            out_ref[...] = jnp.zeros((1, 1), jnp.float32)

        @pl.when(j == 0)
        def _():
            z = jnp.zeros((1, 512), jnp.float32)
            r11[ri, :] = z
            r12[ri, :] = z
            r22[ri, :] = z

        @pl.when(i == 0)
        def _():
            c12[rj, :] = jnp.zeros((1, 512), jnp.float32)

        e11t = jnp.exp2(_dot_t(n1s_j[...], n1b_i[...]).astype(jnp.bfloat16))
        e22t = jnp.exp2(_dot_t(n2s_j[...], n2b_i[...]).astype(jnp.bfloat16))
        e12t = jnp.exp2(_dot_t(n2s_j[...], n1b_i[...]).astype(jnp.bfloat16))

        ones_r = jnp.ones((8, 512), jnp.bfloat16)
        r11[ri, :] = r11[ri, :] + _dot_n(ones_r, e11t)[0:1, :]
        r12[ri, :] = r12[ri, :] + _dot_n(ones_r, e12t)[0:1, :]
        r22[ri, :] = r22[ri, :] + _dot_n(ones_r, e22t)[0:1, :]
        c12[rj, :] = c12[rj, :] + _dot_t(ones_r, e12t)[0:1, :]

        @pl.when(i == j)
        def _():
            dm = (lax.broadcasted_iota(jnp.int32, (512, 512), 0) ==
                  lax.broadcasted_iota(jnp.int32, (512, 512), 1))
            zb16 = jnp.bfloat16(0.0)
            d11[ri, :] = _dot_n(ones_r, jnp.where(dm, e11t, zb16))[0:1, :]
            d12[ri, :] = _dot_n(ones_r, jnp.where(dm, e12t, zb16))[0:1, :]
            d22[ri, :] = _dot_n(ones_r, jnp.where(dm, e22t, zb16))[0:1, :]

        @pl.when((i == _NB - 1) & (j == _NB - 1))
        def _():
            rowmask = (lax.broadcasted_iota(jnp.int32, (_NB, 512), 0) * 512 +
                       lax.broadcasted_iota(jnp.int32, (_NB, 512), 1)) < _N
            x1 = (r11[...] - padc) + (r12[...] - padc) - d11[...]
            x2 = (r22[...] - padc) + (c12[...] - padc) - d22[...]
            li = -jnp.log(d12[...]) + 0.5 * (jnp.log(x1) + jnp.log(x2))
            li = jnp.where(rowmask, li, 0.0)
            out_ref[...] = (jnp.sum(li) / _N).reshape(1, 1)

    return pl.pallas_call(
        body,
        grid=(_NB, _NB),
        in_specs=[
            pl.BlockSpec((512, 256), lambda i, j: (j, 0)),
            pl.BlockSpec((512, 256), lambda i, j: (i, 0)),
            pl.BlockSpec((512, 256), lambda i, j: (j, 0)),
            pl.BlockSpec((512, 256), lambda i, j: (i, 0)),
        ],
        out_specs=pl.BlockSpec((1, 1), lambda i, j: (0, 0)),
        out_shape=jax.ShapeDtypeStruct((1, 1), jnp.float32),
        scratch_shapes=[pltpu.VMEM((_NB, 512), jnp.float32)
                        for _ in range(7)],
    )(n1s, n1b, n2s, n2b)


def kernel(h1, h2, g1, g2, W0, b0, W1, b1, fc1_w, fc1_b, fc2_w, fc2_b):
    s1 = g1[0].reshape(_CH, 128)
    d1 = g1[1].reshape(_CH, 128)
    s2 = g2[0].reshape(_CH, 128)
    d2 = g2[1].reshape(_CH, 128)
    pad = ((0, 30), (0, 0))
    s1p, d1p = jnp.pad(s1, pad), jnp.pad(d1, pad)
    s2p, d2p = jnp.pad(s2, pad), jnp.pad(d2, pad)

    od1, id1, od2, id2 = _sc_degrees(s1, d1, s2, d2)
    od1n = od1[:_N].reshape(_N, 1)
    od2n = od2[:_N].reshape(_N, 1)
    od1p = od1.reshape(_NPAD, 1)
    od2p = od2.reshape(_NPAD, 1)
    id1p = id1.reshape(_NPAD, 1)
    id2p = id2.reshape(_NPAD, 1)

    xs1 = _tc_scale(h1, od1n)
    xs2 = _tc_scale(h2, od2n)

    p1lo, p1hi, p2lo, p2hi = _sc_segsum(
        xs1.reshape(2 * _N, 128), xs2.reshape(2 * _N, 128),
        s1p, d1p, s2p, d2p)

    b0r = b0.reshape(1, 2 * 256)
    m1 = _tc_conv01(p1lo, p1hi, id1p, od1p, W0, b0r, W1)
    m2 = _tc_conv01(p2lo, p2hi, id2p, od2p, W0, b0r, W1)

    q1lo, q1hi, q2lo, q2hi = _sc_segsum(
        m1.reshape(2 * _NPAD, 128), m2.reshape(2 * _NPAD, 128),
        s1p, d1p, s2p, d2p)

    b1r = b1.reshape(1, 256)
    c1r = fc1_b.reshape(1, 128)
    c2r = fc2_b.reshape(1, 256)
    n1b, n1s = _tc_proj(q1lo, q1hi, id1p, b1r, fc1_w, c1r, fc2_w, c2r)
    n2b, n2s = _tc_proj(q2lo, q2hi, id2p, b1r, fc1_w, c1r, fc2_w, c2r)

    res = _tc_loss(n1b, n1s, n2b, n2s)
    return res.reshape(())


# speedup vs baseline: 1.0771x; 1.0428x over previous
import functools

import jax
import jax.numpy as jnp
from jax import lax
from jax.experimental import pallas as pl
from jax.experimental.pallas import tpu as pltpu
from jax.experimental.pallas import tpu_sc as plsc

_N = 10000
_NPAD = 10240
_E = 160000
_CH = _E // 128
_NS = 16
_NT = (_CH + _NS - 1) // _NS
_W = _NPAD // _NS
_NB = _NPAD // 512


def _sc_mesh():
    return plsc.VectorSubcoreMesh(core_axis_name="c", subcore_axis_name="s")


def _sc_degrees(s1, d1, s2, d2):
    out = jax.ShapeDtypeStruct((_NPAD,), jnp.float32)

    @functools.partial(
        pl.kernel,
        out_type=(out, out, out, out),
        mesh=_sc_mesh(),
        scratch_types=[
            pltpu.VMEM((1, 128), jnp.int32),
            pltpu.VMEM((1, 128), jnp.int32),
            pltpu.VMEM((128,), jnp.float32),
            pltpu.VMEM((_W,), jnp.float32),
            pltpu.VMEM_SHARED((_NPAD,), jnp.float32),
            pltpu.VMEM_SHARED((_NPAD,), jnp.float32),
        ],
    )
    def k(s1h, d1h, s2h, d2h, od1, id1, od2, id2, sv, dv, ones_v, zv, hs, hd):
        c = lax.axis_index("c")
        s = lax.axis_index("s")
        for u in range(8):
            ones_v[pl.ds(u * 16, 16)] = jnp.ones((16,), jnp.float32)

        def zf(i, _):
            zv[pl.ds(i * 16, 16)] = jnp.zeros((16,), jnp.float32)
            return 0

        lax.fori_loop(0, _W // 16, zf, 0)
        pltpu.sync_copy(zv, hs.at[pl.ds(s * _W, _W)])
        pltpu.sync_copy(zv, hd.at[pl.ds(s * _W, _W)])
        plsc.subcore_barrier()

        def run(src_hbm, dst_hbm):
            def body(t, _):
                cid = s + t * _NS

                @pl.when(cid < _CH)
                def _():
                    pltpu.sync_copy(src_hbm.at[pl.ds(cid, 1)], sv)
                    pltpu.sync_copy(dst_hbm.at[pl.ds(cid, 1)], dv)
                    pltpu.sync_copy(ones_v, hs.at[sv.at[0]], add=True)
                    pltpu.sync_copy(ones_v, hd.at[dv.at[0]], add=True)

                return 0

            lax.fori_loop(0, _NT, body, 0)

        @pl.when(c == 0)
        def _():
            run(s1h, d1h)

        @pl.when(c == 1)
        def _():
            run(s2h, d2h)

        plsc.subcore_barrier()
        sl = pl.ds(s * _W, _W)

        @pl.when(c == 0)
        def _():
            pltpu.sync_copy(hs.at[sl], od1.at[sl])
            pltpu.sync_copy(hd.at[sl], id1.at[sl])

        @pl.when(c == 1)
        def _():
            pltpu.sync_copy(hs.at[sl], od2.at[sl])
            pltpu.sync_copy(hd.at[sl], id2.at[sl])

    return k(s1, d1, s2, d2)


def _sc_segsum(x1r, x2r, s1, d1, s2, d2):
    out = jax.ShapeDtypeStruct((_NPAD, 128), jnp.float32)

    nbuf = 2

    @functools.partial(
        pl.kernel,
        out_type=(out, out, out, out),
        mesh=_sc_mesh(),
        scratch_types=[
            pltpu.VMEM((8, 128), jnp.int32),
            pltpu.VMEM((8, 128), jnp.int32),
            [pltpu.VMEM((128,), jnp.int32) for _ in range(nbuf)],
            [pltpu.VMEM((128, 128), jnp.float32) for _ in range(nbuf)],
            pltpu.VMEM((64, 128), jnp.float32),
            pltpu.VMEM_SHARED((_NPAD, 128), jnp.float32),
            pltpu.SemaphoreType.DMA,
        ],
    )
    def k(x1h, x2h, s1h, d1h, s2h, d2h, o1lo, o1hi, o2lo, o2hi,
          sv8, dv8, gvs, rbs, zb, acc, gsem):
        c = lax.axis_index("c")
        s = lax.axis_index("s")
        base = s * 80
        nt = jnp.minimum(80, _CH - base)

        def zf(i, _):
            for u in range(8):
                zb[i, pl.ds(u * 16, 16)] = jnp.zeros((16,), jnp.float32)
            return 0

        lax.fori_loop(0, 64, zf, 0)

        def one_view(x_hbm, src_hbm, dst_hbm, out_lo, out_hi):
            for kk in range(_W // 64):
                pltpu.sync_copy(zb, acc.at[pl.ds(s * _W + kk * 64, 64)])
            plsc.subcore_barrier()

            def body(o, _):
                o8 = base + o * 8
                pltpu.sync_copy(src_hbm.at[pl.ds(o8, 8)], sv8)
                pltpu.sync_copy(dst_hbm.at[pl.ds(o8, 8)], dv8)
                for gp in range(4):
                    ks = [gp * nbuf + b for b in range(nbuf)]
                    for b in range(nbuf):
                        @pl.when(o * 8 + ks[b] < nt)
                        def _(b=b, kk=ks[b]):
                            for u in range(8):
                                gvs[b][pl.ds(u * 16, 16)] = (
                                    sv8[kk, pl.ds(u * 16, 16)] * 2 + c)
                            pltpu.async_copy(x_hbm.at[gvs[b]], rbs[b], gsem)
                    for b in range(nbuf):
                        @pl.when(o * 8 + ks[b] < nt)
                        def _(b=b, kk=ks[b]):
                            pltpu.make_async_copy(
                                x_hbm.at[gvs[b]], rbs[b], gsem).wait()
                            pltpu.sync_copy(
                                rbs[b], acc.at[dv8.at[kk]], add=True)
                return 0

            lax.fori_loop(0, 10, body, 0)
            plsc.subcore_barrier()
            sl = pl.ds(s * _W, _W)

            @pl.when(c == 0)
            def _():
                pltpu.sync_copy(acc.at[sl], out_lo.at[sl])

            @pl.when(c == 1)
            def _():
                pltpu.sync_copy(acc.at[sl], out_hi.at[sl])

            plsc.subcore_barrier()

        one_view(x1h, s1h, d1h, o1lo, o1hi)
        one_view(x2h, s2h, d2h, o2lo, o2hi)

    return k(x1r, x2r, s1, d1, s2, d2)


def _tc_scale(h, odeg):
    def body(h_ref, od_ref, o_ref):
        scale = lax.rsqrt(jnp.maximum(od_ref[...], 1.0))
        o_ref[...] = h_ref[...] * scale

    return pl.pallas_call(
        body,
        grid=(10,),
        in_specs=[
            pl.BlockSpec((1000, 256), lambda i: (i, 0)),
            pl.BlockSpec((1000, 1), lambda i: (i, 0)),
        ],
        out_specs=pl.BlockSpec((1000, 256), lambda i: (i, 0)),
        out_shape=jax.ShapeDtypeStruct((_N, 256), jnp.float32),
    )(h, odeg)


def _tc_conv01(plo, phi, ideg, odeg, W0, b0, W1):
    def body(plo_ref, phi_ref, id_ref, od_ref, w0_ref, b0_ref, w1_ref, o_ref):
        a = jnp.dot(plo_ref[...], w0_ref[0:128, :],
                    preferred_element_type=jnp.float32)
        a = a + jnp.dot(phi_ref[...], w0_ref[128:256, :],
                        preferred_element_type=jnp.float32)
        iscale = lax.rsqrt(jnp.maximum(id_ref[...], 1.0))
        h = jnp.maximum(a * iscale + b0_ref[...], 0.0)
        oscale = lax.rsqrt(jnp.maximum(od_ref[...], 1.0))
        o_ref[...] = jnp.dot(h * oscale, w1_ref[...],
                             preferred_element_type=jnp.float32)

    return pl.pallas_call(
        body,
        grid=(_NB,),
        in_specs=[
            pl.BlockSpec((512, 128), lambda i: (i, 0)),
            pl.BlockSpec((512, 128), lambda i: (i, 0)),
            pl.BlockSpec((512, 1), lambda i: (i, 0)),
            pl.BlockSpec((512, 1), lambda i: (i, 0)),
            pl.BlockSpec((256, 512), lambda i: (0, 0)),
            pl.BlockSpec((1, 512), lambda i: (0, 0)),
            pl.BlockSpec((512, 256), lambda i: (0, 0)),
        ],
        out_specs=pl.BlockSpec((512, 256), lambda i: (i, 0)),
        out_shape=jax.ShapeDtypeStruct((_NPAD, 256), jnp.float32),
    )(plo, phi, ideg, odeg, W0, b0, W1)


def _tc_proj(qlo, qhi, ideg, b1, fc1_w, fc1_b, fc2_w, fc2_b):
    def body(qlo_ref, qhi_ref, id_ref, b1_ref, w1_ref, c1_ref, w2_ref, c2_ref,
             ob_ref, o2b_ref):
        i = pl.program_id(0)
        p = jnp.concatenate([qlo_ref[...], qhi_ref[...]], axis=1)
        iscale = lax.rsqrt(jnp.maximum(id_ref[...], 1.0))
        e = jnp.maximum(p * iscale + b1_ref[...], 0.0)
        t = jnp.dot(e, w1_ref[...], preferred_element_type=jnp.float32)
        t = t + c1_ref[...]
        t = jnp.where(t > 0, t, jnp.exp(t) - 1.0)
        z = jnp.dot(t, w2_ref[...], preferred_element_type=jnp.float32)
        z = z + c2_ref[...]
        nrm = jnp.sqrt(jnp.sum(z * z, axis=1, keepdims=True))
        n = z / jnp.maximum(nrm, 1e-12)
        row = i * 512 + lax.broadcasted_iota(jnp.int32, (512, 1), 0)
        n = jnp.where(row < _N, n, 0.0)
        ob_ref[...] = n.astype(jnp.bfloat16)
        o2b_ref[...] = (n * 2.8853900817779268).astype(jnp.bfloat16)

    return pl.pallas_call(
        body,
        grid=(_NB,),
        in_specs=[
            pl.BlockSpec((512, 128), lambda i: (i, 0)),
            pl.BlockSpec((512, 128), lambda i: (i, 0)),
            pl.BlockSpec((512, 1), lambda i: (i, 0)),
            pl.BlockSpec((1, 256), lambda i: (0, 0)),
            pl.BlockSpec((256, 128), lambda i: (0, 0)),
            pl.BlockSpec((1, 128), lambda i: (0, 0)),
            pl.BlockSpec((128, 256), lambda i: (0, 0)),
            pl.BlockSpec((1, 256), lambda i: (0, 0)),
        ],
        out_specs=[
            pl.BlockSpec((512, 256), lambda i: (i, 0)),
            pl.BlockSpec((512, 256), lambda i: (i, 0)),
        ],
        out_shape=(
            jax.ShapeDtypeStruct((_NPAD, 256), jnp.bfloat16),
            jax.ShapeDtypeStruct((_NPAD, 256), jnp.bfloat16),
        ),
    )(qlo, qhi, ideg, b1, fc1_w, fc1_b, fc2_w, fc2_b)


def _dot_t(x, y):
    return lax.dot_general(
        x, y, (((1,), (1,)), ((), ())), preferred_element_type=jnp.float32)


def _dot_n(x, y):
    return lax.dot_general(
        x, y, (((1,), (0,)), ((), ())), preferred_element_type=jnp.float32)


def _tc_loss(n1b, n1s, n2b, n2s):
    padc = float(_NPAD - _N)

    def body(n1s_j, n1b_i, n2s_j, n2b_i,
             out_ref, r11, r12, r22, c12, d11, d22, d12):
        i = pl.program_id(0)
        j = pl.program_id(1)

        ri = pl.ds(i, 1)
        rj = pl.ds(j, 1)

        @pl.when((i == 0) & (j == 0))
        def _():
            out_ref[...] = jnp.zeros((1, 1), jnp.float32)

        @pl.when(j == 0)
        def _():
            z = jnp.zeros((1, 512), jnp.float32)
            r11[ri, :] = z
            r12[ri, :] = z
            r22[ri, :] = z

        @pl.when(i == 0)
        def _():
            c12[rj, :] = jnp.zeros((1, 512), jnp.float32)

        e11t = jnp.exp2(_dot_t(n1s_j[...], n1b_i[...])).astype(jnp.bfloat16)
        e22t = jnp.exp2(_dot_t(n2s_j[...], n2b_i[...])).astype(jnp.bfloat16)
        e12t = jnp.exp2(_dot_t(n2s_j[...], n1b_i[...])).astype(jnp.bfloat16)

        ones_r = jnp.ones((8, 512), jnp.bfloat16)
        r11[ri, :] = r11[ri, :] + _dot_n(ones_r, e11t)[0:1, :]
        r12[ri, :] = r12[ri, :] + _dot_n(ones_r, e12t)[0:1, :]
        r22[ri, :] = r22[ri, :] + _dot_n(ones_r, e22t)[0:1, :]
        c12[rj, :] = c12[rj, :] + _dot_t(ones_r, e12t)[0:1, :]

        @pl.when(i == j)
        def _():
            dm = (lax.broadcasted_iota(jnp.int32, (512, 512), 0) ==
                  lax.broadcasted_iota(jnp.int32, (512, 512), 1))
            zb16 = jnp.bfloat16(0.0)
            d11[ri, :] = _dot_n(ones_r, jnp.where(dm, e11t, zb16))[0:1, :]
            d12[ri, :] = _dot_n(ones_r, jnp.where(dm, e12t, zb16))[0:1, :]
            d22[ri, :] = _dot_n(ones_r, jnp.where(dm, e22t, zb16))[0:1, :]

        @pl.when((i == _NB - 1) & (j == _NB - 1))
        def _():
            rowmask = (lax.broadcasted_iota(jnp.int32, (_NB, 512), 0) * 512 +
                       lax.broadcasted_iota(jnp.int32, (_NB, 512), 1)) < _N
            x1 = (r11[...] - padc) + (r12[...] - padc) - d11[...]
            x2 = (r22[...] - padc) + (c12[...] - padc) - d22[...]
            li = -jnp.log(d12[...]) + 0.5 * (jnp.log(x1) + jnp.log(x2))
            li = jnp.where(rowmask, li, 0.0)
            out_ref[...] = (jnp.sum(li) / _N).reshape(1, 1)

    return pl.pallas_call(
        body,
        grid=(_NB, _NB),
        in_specs=[
            pl.BlockSpec((512, 256), lambda i, j: (j, 0)),
            pl.BlockSpec((512, 256), lambda i, j: (i, 0)),
            pl.BlockSpec((512, 256), lambda i, j: (j, 0)),
            pl.BlockSpec((512, 256), lambda i, j: (i, 0)),
        ],
        out_specs=pl.BlockSpec((1, 1), lambda i, j: (0, 0)),
        out_shape=jax.ShapeDtypeStruct((1, 1), jnp.float32),
        scratch_shapes=[pltpu.VMEM((_NB, 512), jnp.float32)
                        for _ in range(7)],
    )(n1s, n1b, n2s, n2b)


def kernel(h1, h2, g1, g2, W0, b0, W1, b1, fc1_w, fc1_b, fc2_w, fc2_b):
    s1 = g1[0].reshape(_CH, 128)
    d1 = g1[1].reshape(_CH, 128)
    s2 = g2[0].reshape(_CH, 128)
    d2 = g2[1].reshape(_CH, 128)
    pad = ((0, 30), (0, 0))
    s1p, d1p = jnp.pad(s1, pad), jnp.pad(d1, pad)
    s2p, d2p = jnp.pad(s2, pad), jnp.pad(d2, pad)

    od1, id1, od2, id2 = _sc_degrees(s1, d1, s2, d2)
    od1n = od1[:_N].reshape(_N, 1)
    od2n = od2[:_N].reshape(_N, 1)
    od1p = od1.reshape(_NPAD, 1)
    od2p = od2.reshape(_NPAD, 1)
    id1p = id1.reshape(_NPAD, 1)
    id2p = id2.reshape(_NPAD, 1)

    xs1 = _tc_scale(h1, od1n)
    xs2 = _tc_scale(h2, od2n)

    p1lo, p1hi, p2lo, p2hi = _sc_segsum(
        xs1.reshape(2 * _N, 128), xs2.reshape(2 * _N, 128),
        s1p, d1p, s2p, d2p)

    b0r = b0.reshape(1, 2 * 256)
    m1 = _tc_conv01(p1lo, p1hi, id1p, od1p, W0, b0r, W1)
    m2 = _tc_conv01(p2lo, p2hi, id2p, od2p, W0, b0r, W1)

    q1lo, q1hi, q2lo, q2hi = _sc_segsum(
        m1.reshape(2 * _NPAD, 128), m2.reshape(2 * _NPAD, 128),
        s1p, d1p, s2p, d2p)

    b1r = b1.reshape(1, 256)
    c1r = fc1_b.reshape(1, 128)
    c2r = fc2_b.reshape(1, 256)
    n1b, n1s = _tc_proj(q1lo, q1hi, id1p, b1r, fc1_w, c1r, fc2_w, c2r)
    n2b, n2s = _tc_proj(q2lo, q2hi, id2p, b1r, fc1_w, c1r, fc2_w, c2r)

    res = _tc_loss(n1b, n1s, n2b, n2s)
    return res.reshape(())
